# baseline (speedup 1.0000x reference)
import functools

import jax
import jax.numpy as jnp
from jax import lax
from jax.experimental import pallas as pl
from jax.experimental.pallas import tpu as pltpu
from jax.experimental.pallas import tpu_sc as plsc


def _emb_call(B, D, NC, NS):
    NW = NC * NS
    b_per_w = B // NW
    L = 16
    G = b_per_w // L
    mesh = plsc.VectorSubcoreMesh(core_axis_name="c", subcore_axis_name="s")

    @functools.partial(
        pl.kernel,
        mesh=mesh,
        out_type=jax.ShapeDtypeStruct((B, D), jnp.float32),
        compiler_params=pltpu.CompilerParams(use_tc_tiling_on_sc=False),
        scratch_types=[
            pltpu.VMEM((b_per_w,), jnp.int32),
            pltpu.SemaphoreType.DMA,
        ],
    )
    def emb(idx_hbm, table_hbm, out_hbm, idx_v, sem):
        wid = lax.axis_index("s") * NC + lax.axis_index("c")
        base = wid * b_per_w
        pltpu.sync_copy(idx_hbm.at[pl.ds(base, b_per_w)], idx_v)
        lanes = lax.broadcasted_iota(jnp.int32, (L,), 0)

        def group(g, carry):
            vec = idx_v[pl.ds(g * L, L)]
            for l in range(L):
                val = vec[l]
                pltpu.async_copy(
                    table_hbm.at[pl.ds(val, 1)],
                    out_hbm.at[pl.ds(base + g * L + l, 1)],
                    sem,
                )
            pltpu.make_async_copy(
                table_hbm.at[pl.ds(0, L)], out_hbm.at[pl.ds(0, L)], sem
            ).wait()
            return carry

        lax.fori_loop(0, G, group, 0)

    return emb


def kernel(input_sentences, embedding_weight):
    S0, S1 = input_sentences.shape
    V, D = embedding_weight.shape
    B = S0 * S1
    info = plsc.get_sparse_core_info()
    NC, NS = info.num_cores, info.num_subcores
    idx = input_sentences.reshape(B).astype(jnp.int32)
    out = _emb_call(B, D, NC, NS)(idx, embedding_weight)
    return out.reshape(S0, S1, D)
